# Initial kernel scaffold; baseline (speedup 1.0000x reference)
import functools

import jax
import jax.numpy as jnp
from jax import lax
from jax.experimental import pallas as pl
from jax.experimental.pallas import tpu as pltpu
from jax.experimental.pallas import tpu_sc as plsc

N = 10000
E = 320000
D = 128
EPS = 1e-7

NC = 2
NS = 16
LK = 16

K = 128
CHUNKS = -(-E // (NS * K))
EPT = CHUNKS * K
EPAD = EPT * NS
RPT = N // NS
RCH = 5
RCHROWS = RPT // RCH


def _norm(x):
    return jnp.maximum(jnp.sqrt(jnp.sum(x * x, axis=-1, keepdims=True)), EPS)


def _artanh(x):
    return jnp.arctanh(jnp.clip(x, -1.0 + 1e-6, 1.0 - 1e-6))


def _mobius_scalar_mul(r, x):
    n = _norm(x)
    return jnp.tanh(r * _artanh(n)) * x / n


def _mobius_addition(x, y):
    xy = jnp.sum(x * y, axis=-1, keepdims=True)
    x2 = jnp.sum(x * x, axis=-1, keepdims=True)
    y2 = jnp.sum(y * y, axis=-1, keepdims=True)
    num = (1.0 + 2.0 * xy + y2) * x + (1.0 - x2) * y
    den = jnp.maximum(1.0 + 2.0 * xy + x2 * y2, EPS)
    return num / den


_ROWS_BLK = 2000


def _pre_body(lx_ref, tan_ref):
    x = lx_ref[...]
    n = _norm(x)
    tan_ref[...] = _artanh(n) * x / n


def _pre_tc(lorentz_x):
    return pl.pallas_call(
        _pre_body,
        out_shape=jax.ShapeDtypeStruct((N, D), jnp.float32),
        grid=(N // _ROWS_BLK,),
        in_specs=[pl.BlockSpec((_ROWS_BLK, D), lambda i: (i, 0))],
        out_specs=pl.BlockSpec((_ROWS_BLK, D), lambda i: (i, 0)),
    )(lorentz_x)


def _post_body(agge_ref, aggt_ref, ex_ref, lx_ref, eo_ref, lo_ref):
    eo_ref[...] = 0.5 * agge_ref[...] + 0.5 * ex_ref[...]
    t = aggt_ref[...]
    n = _norm(t)
    lorentz_pre = jnp.tanh(n) * t / n
    l_skip = _mobius_scalar_mul(0.5, lx_ref[...])
    l_out = _mobius_scalar_mul(0.5, lorentz_pre)
    lo_ref[...] = _mobius_addition(l_out, l_skip)


def _post_tc(agg_e, agg_t, euclidean_x, lorentz_x):
    blk = pl.BlockSpec((_ROWS_BLK, D), lambda i: (i, 0))
    return pl.pallas_call(
        _post_body,
        out_shape=(
            jax.ShapeDtypeStruct((N, D), jnp.float32),
            jax.ShapeDtypeStruct((N, D), jnp.float32),
        ),
        grid=(N // _ROWS_BLK,),
        in_specs=[blk, blk, blk, blk],
        out_specs=(blk, blk),
    )(agg_e, agg_t, euclidean_x, lorentz_x)


def _sc_spmm_body(xcat_hbm, dst_hbm, src_hbm, val_hbm, out_hbm,
                  idx_v, dst_v, val_v, rows_v, acc_sh, sem):
    c = lax.axis_index("c")
    s = lax.axis_index("s")
    zero16 = jnp.zeros((LK,), jnp.float32)

    def zrow(r, carry):
        for j in range(D // LK):
            rows_v[r, pl.ds(j * LK, LK)] = zero16
        return carry

    lax.fori_loop(0, K, zrow, 0)
    for r in range(RCH):
        pltpu.sync_copy(
            rows_v.at[pl.ds(0, RCHROWS)],
            acc_sh.at[pl.ds(s * RPT + r * RCHROWS, RCHROWS)],
        )
    plsc.subcore_barrier()

    row_off = c * N

    def chunk(g, carry):
        e0 = s * EPT + g * K
        pltpu.sync_copy(src_hbm.at[pl.ds(e0, K)], idx_v)
        pltpu.sync_copy(dst_hbm.at[pl.ds(e0, K)], dst_v)
        pltpu.sync_copy(val_hbm.at[pl.ds(e0, K)], val_v)
        for j in range(K // LK):
            idx_v[pl.ds(j * LK, LK)] = idx_v[pl.ds(j * LK, LK)] + row_off
        pltpu.async_copy(xcat_hbm.at[idx_v], rows_v, sem).wait()

        def scale(e, inner):
            v = val_v[e]
            for j in range(D // LK):
                rows_v[e, pl.ds(j * LK, LK)] = rows_v[e, pl.ds(j * LK, LK)] * v
            return inner

        lax.fori_loop(0, K, scale, 0)
        pltpu.sync_copy(rows_v, acc_sh.at[dst_v], add=True)
        return carry

    lax.fori_loop(0, CHUNKS, chunk, 0)
    plsc.subcore_barrier()

    pltpu.sync_copy(
        acc_sh.at[pl.ds(s * RPT, RPT)],
        out_hbm.at[pl.ds(row_off + s * RPT, RPT)],
    )


def _sc_spmm(xcat, dst, src, val):
    mesh = plsc.VectorSubcoreMesh(
        core_axis_name="c", subcore_axis_name="s", num_cores=NC, num_subcores=NS
    )
    f = pl.kernel(
        _sc_spmm_body,
        out_type=jax.ShapeDtypeStruct((NC * N, D), jnp.float32),
        mesh=mesh,
        scratch_types=[
            pltpu.VMEM((K,), jnp.int32),
            pltpu.VMEM((K,), jnp.int32),
            pltpu.VMEM((K,), jnp.float32),
            pltpu.VMEM((K, D), jnp.float32),
            pltpu.VMEM_SHARED((N, D), jnp.float32),
            pltpu.SemaphoreType.DMA,
        ],
    )
    return f(xcat, dst, src, val)


def kernel(euclidean_x, lorentz_x, adj_indices, adj_values):
    tangent_x = _pre_tc(lorentz_x)
    xcat = jnp.concatenate([euclidean_x, tangent_x], axis=0)
    pad = EPAD - E
    dst = jnp.concatenate([adj_indices[0], jnp.zeros((pad,), jnp.int32)])
    src = jnp.concatenate([adj_indices[1], jnp.zeros((pad,), jnp.int32)])
    val = jnp.concatenate([adj_values, jnp.zeros((pad,), jnp.float32)])
    agg = _sc_spmm(xcat, dst, src, val)
    return _post_tc(agg[:N], agg[N:], euclidean_x, lorentz_x)


# speedup vs baseline: 4.3221x; 4.3221x over previous
import functools

import jax
import jax.numpy as jnp
from jax import lax
from jax.experimental import pallas as pl
from jax.experimental.pallas import tpu as pltpu
from jax.experimental.pallas import tpu_sc as plsc

N = 10000
E = 320000
D = 128
EPS = 1e-7

NC = 2
NS = 16
LK = 16

K = 128
CHUNKS = -(-E // (NS * K))
EPT = CHUNKS * K
EPAD = EPT * NS
RPT = 632
NPAD = RPT * NS
RCHS = (128, 128, 128, 128, 120)


def _norm(x):
    return jnp.maximum(jnp.sqrt(jnp.sum(x * x, axis=-1, keepdims=True)), EPS)


def _artanh(x):
    x = jnp.clip(x, -1.0 + 1e-6, 1.0 - 1e-6)
    return 0.5 * jnp.log((1.0 + x) / (1.0 - x))


def _mobius_scalar_mul(r, x):
    n = _norm(x)
    return jnp.tanh(r * _artanh(n)) * x / n


def _mobius_addition(x, y):
    xy = jnp.sum(x * y, axis=-1, keepdims=True)
    x2 = jnp.sum(x * x, axis=-1, keepdims=True)
    y2 = jnp.sum(y * y, axis=-1, keepdims=True)
    num = (1.0 + 2.0 * xy + y2) * x + (1.0 - x2) * y
    den = jnp.maximum(1.0 + 2.0 * xy + x2 * y2, EPS)
    return num / den


_ROWS_BLK = 2000


def _pre_body(lx_ref, tan_ref):
    x = lx_ref[...]
    n = _norm(x)
    tan_ref[...] = _artanh(n) * x / n


def _pre_tc(lorentz_x):
    return pl.pallas_call(
        _pre_body,
        out_shape=jax.ShapeDtypeStruct((N, D), jnp.float32),
        grid=(N // _ROWS_BLK,),
        in_specs=[pl.BlockSpec((_ROWS_BLK, D), lambda i: (i, 0))],
        out_specs=pl.BlockSpec((_ROWS_BLK, D), lambda i: (i, 0)),
    )(lorentz_x)


def _post_body(agge_ref, aggt_ref, ex_ref, lx_ref, eo_ref, lo_ref):
    eo_ref[...] = 0.5 * agge_ref[...] + 0.5 * ex_ref[...]
    t = aggt_ref[...]
    n = _norm(t)
    lorentz_pre = jnp.tanh(n) * t / n
    l_skip = _mobius_scalar_mul(0.5, lx_ref[...])
    l_out = _mobius_scalar_mul(0.5, lorentz_pre)
    lo_ref[...] = _mobius_addition(l_out, l_skip)


def _post_tc(agg_e, agg_t, euclidean_x, lorentz_x):
    blk = pl.BlockSpec((_ROWS_BLK, D), lambda i: (i, 0))
    return pl.pallas_call(
        _post_body,
        out_shape=(
            jax.ShapeDtypeStruct((N, D), jnp.float32),
            jax.ShapeDtypeStruct((N, D), jnp.float32),
        ),
        grid=(N // _ROWS_BLK,),
        in_specs=[blk, blk, blk, blk],
        out_specs=(blk, blk),
    )(agg_e, agg_t, euclidean_x, lorentz_x)


def _sc_spmm_body(xcat_hbm, dst_hbm, src_hbm, val_hbm, out_hbm,
                  idx_v, dst_v, val_v, rows_v, acc_sh, sem):
    c = lax.axis_index("c")
    s = lax.axis_index("s")
    zero16 = jnp.zeros((LK,), jnp.float32)

    def zrow(r, carry):
        for j in range(D // LK):
            rows_v[r, pl.ds(j * LK, LK)] = zero16
        return carry

    lax.fori_loop(0, K, zrow, 0)
    off = 0
    for sz in RCHS:
        pltpu.sync_copy(
            rows_v.at[pl.ds(0, sz)],
            acc_sh.at[pl.ds(s * RPT + off, sz)],
        )
        off += sz
    plsc.subcore_barrier()

    row_off = c * N

    def chunk(g, carry):
        e0 = s * EPT + g * K
        pltpu.sync_copy(src_hbm.at[pl.ds(e0, K)], idx_v)
        pltpu.sync_copy(dst_hbm.at[pl.ds(e0, K)], dst_v)
        pltpu.sync_copy(val_hbm.at[pl.ds(e0, K)], val_v)
        for j in range(K // LK):
            idx_v[pl.ds(j * LK, LK)] = idx_v[pl.ds(j * LK, LK)] + row_off
        pltpu.async_copy(xcat_hbm.at[idx_v], rows_v, sem).wait()

        def scale(t, inner):
            vals16 = val_v[pl.ds(t * LK, LK)]
            for el in range(LK):
                e = t * LK + el
                v = vals16[el]
                for j in range(D // LK):
                    rows_v[e, pl.ds(j * LK, LK)] = (
                        rows_v[e, pl.ds(j * LK, LK)] * v
                    )
            return inner

        lax.fori_loop(0, K // LK, scale, 0)
        pltpu.sync_copy(rows_v, acc_sh.at[dst_v], add=True)
        return carry

    lax.fori_loop(0, CHUNKS, chunk, 0)
    plsc.subcore_barrier()

    off = 0
    for sz in RCHS:
        pltpu.sync_copy(
            acc_sh.at[pl.ds(s * RPT + off, sz)],
            out_hbm.at[pl.ds(c * NPAD + s * RPT + off, sz)],
        )
        off += sz


def _sc_spmm(xcat, dst, src, val):
    mesh = plsc.VectorSubcoreMesh(
        core_axis_name="c", subcore_axis_name="s", num_cores=NC, num_subcores=NS
    )
    f = pl.kernel(
        _sc_spmm_body,
        out_type=jax.ShapeDtypeStruct((NC * NPAD, D), jnp.float32),
        mesh=mesh,
        scratch_types=[
            pltpu.VMEM((K,), jnp.int32),
            pltpu.VMEM((K,), jnp.int32),
            pltpu.VMEM((K,), jnp.float32),
            pltpu.VMEM((K, D), jnp.float32),
            pltpu.VMEM_SHARED((NPAD, D), jnp.float32),
            pltpu.SemaphoreType.DMA,
        ],
    )
    return f(xcat, dst, src, val)


def kernel(euclidean_x, lorentz_x, adj_indices, adj_values):
    tangent_x = _pre_tc(lorentz_x)
    xcat = jnp.concatenate([euclidean_x, tangent_x], axis=0)
    pad = EPAD - E
    dst = jnp.concatenate([adj_indices[0], jnp.zeros((pad,), jnp.int32)])
    src = jnp.concatenate([adj_indices[1], jnp.zeros((pad,), jnp.int32)])
    val = jnp.concatenate([adj_values, jnp.zeros((pad,), jnp.float32)])
    agg = _sc_spmm(xcat, dst, src, val)
    return _post_tc(agg[:N], agg[NPAD:NPAD + N], euclidean_x, lorentz_x)
